# baseline (speedup 1.0000x reference)
import functools

import jax
import jax.numpy as jnp
from jax import lax
from jax.experimental import pallas as pl
from jax.experimental.pallas import tpu as pltpu
from jax.experimental.pallas import tpu_sc as plsc

B = 16384
D = 64
NC = 2
NS = 16
NW = NC * NS
BPW = B // NW
K = 128
CH = BPW // K


def _sc_body(idx4_hbm, idx5_hbm, cv0_hbm, cv1_hbm, cv2_hbm, cv3_hbm, out_hbm,
             idx4_v, idx5_v, r1_v, r2_v, d0_v, d3_v,
             sem0, sem1, sem2, sem3):
    wid = lax.axis_index("s") * NC + lax.axis_index("c")
    pltpu.sync_copy(idx4_hbm.at[wid], idx4_v)
    pltpu.sync_copy(idx5_hbm.at[wid], idx5_v)
    for j in range(CH):
        cb = wid * BPW + j * K
        g2 = pltpu.async_copy(cv2_hbm.at[idx5_v.at[j]], r2_v, sem2)
        g1 = pltpu.async_copy(cv1_hbm.at[idx4_v.at[j]], r1_v, sem1)
        c0 = pltpu.async_copy(cv0_hbm.at[pl.ds(cb, K)], d0_v, sem0)
        c3 = pltpu.async_copy(cv3_hbm.at[pl.ds(cb, K)], d3_v, sem3)
        g2.wait()
        pltpu.sync_copy(r2_v, out_hbm.at[pl.ds(cb, K), 0])
        c0.wait()
        pltpu.sync_copy(d0_v, out_hbm.at[pl.ds(cb, K), 1])
        g1.wait()
        pltpu.sync_copy(r1_v, out_hbm.at[pl.ds(cb, K), 2])
        c3.wait()
        pltpu.sync_copy(d3_v, out_hbm.at[pl.ds(cb, K), 3])


_sc_call = pl.kernel(
    _sc_body,
    mesh=plsc.VectorSubcoreMesh(core_axis_name="c", subcore_axis_name="s"),
    compiler_params=pltpu.CompilerParams(use_tc_tiling_on_sc=False),
    out_type=jax.ShapeDtypeStruct((B, 4, D), jnp.float32),
    scratch_types=[
        pltpu.VMEM((CH, K), jnp.int32),
        pltpu.VMEM((CH, K), jnp.int32),
        pltpu.VMEM((K, D), jnp.float32),
        pltpu.VMEM((K, D), jnp.float32),
        pltpu.VMEM((K, D), jnp.float32),
        pltpu.VMEM((K, D), jnp.float32),
        pltpu.SemaphoreType.DMA,
        pltpu.SemaphoreType.DMA,
        pltpu.SemaphoreType.DMA,
        pltpu.SemaphoreType.DMA,
    ],
)


@jax.jit
def kernel(adj_list_4, adj_list_5, concat_vecs_0, concat_vecs_1,
           concat_vecs_2, concat_vecs_3):
    idx4 = adj_list_4.astype(jnp.int32).reshape(NW, CH, K)
    idx5 = adj_list_5.astype(jnp.int32).reshape(NW, CH, K)
    out = _sc_call(idx4, idx5, concat_vecs_0, concat_vecs_1,
                   concat_vecs_2, concat_vecs_3)
    return out.reshape(B, 4 * D)
